# baseline (speedup 1.0000x reference)
import functools

import jax
import jax.numpy as jnp
from jax import lax
from jax.experimental import pallas as pl
from jax.experimental.pallas import tpu as pltpu
from jax.experimental.pallas import tpu_sc as plsc

EMBED = 64
NC = 2
NS = 16
NW = NC * NS
CHUNK = 128
NBUF = 3


def _emb_body(idx_hbm, table_hbm, pos2_hbm, out_hbm,
              idx_v, rows_v, pos_sh, sem_g, sem_s, sem_p):
    chunks_per_w = idx_hbm.shape[1]
    t = pos2_hbm.shape[0] // 2
    c = lax.axis_index("c")
    s = lax.axis_index("s")
    w = s * NC + c
    pltpu.sync_copy(idx_hbm.at[w], idx_v)

    @pl.when(s == 0)
    def _():
        pltpu.sync_copy(pos2_hbm, pos_sh)
    plsc.subcore_barrier()

    base = w * (chunks_per_w * CHUNK)

    def prefill(h):
        poff = pl.multiple_of(lax.rem(h * CHUNK, t), 8)
        pltpu.async_copy(pos_sh.at[pl.ds(poff, CHUNK)],
                         rows_v.at[lax.rem(h, NBUF)], sem_p)

    def wait_prefill():
        pltpu.make_async_copy(pos_sh.at[pl.ds(0, CHUNK)], rows_v.at[0],
                              sem_p).wait()

    def wait_store():
        pltpu.make_async_copy(rows_v.at[0], out_hbm.at[pl.ds(base, CHUNK)],
                              sem_s).wait()

    prefill(0)

    def chunk_body(h, _):
        buf = lax.rem(h, NBUF)
        wait_prefill()

        @pl.when(h + 1 < chunks_per_w)
        def _():
            @pl.when(h >= 2)
            def _():
                wait_store()
            prefill(h + 1)

        pltpu.async_copy(
            table_hbm.at[idx_v.at[h]], rows_v.at[buf], sem_g, add=True
        ).wait()
        pltpu.async_copy(
            rows_v.at[buf],
            out_hbm.at[pl.ds(pl.multiple_of(base + h * CHUNK, CHUNK), CHUNK)],
            sem_s,
        )
        return ()

    lax.fori_loop(0, chunks_per_w, chunk_body, ())

    for _ in range(NBUF):
        wait_store()


def kernel(x, token_emb, pos_emb):
    B, Tcur = x.shape
    total = B * Tcur
    chunks_per_w = total // (NW * CHUNK)
    xi = x.astype(jnp.int32).reshape(NW, chunks_per_w, CHUNK)
    tpad = jnp.pad(token_emb, ((0, 0), (0, 128 - EMBED)))
    pos = pos_emb[0, :Tcur, :]
    pos2 = jnp.pad(jnp.concatenate([pos, pos], axis=0),
                   ((0, 0), (0, 128 - EMBED)))

    emb = functools.partial(
        pl.kernel,
        out_type=jax.ShapeDtypeStruct((total, 128), jnp.float32),
        mesh=plsc.VectorSubcoreMesh(core_axis_name="c", subcore_axis_name="s"),
        compiler_params=pltpu.CompilerParams(use_tc_tiling_on_sc=True,
                                             needs_layout_passes=False),
        scratch_types=[
            pltpu.VMEM((chunks_per_w, CHUNK), jnp.int32),
            pltpu.VMEM((NBUF, CHUNK, 128), jnp.float32),
            pltpu.VMEM_SHARED((2 * Tcur, 128), jnp.float32),
            pltpu.SemaphoreType.DMA,
            pltpu.SemaphoreType.DMA,
            pltpu.SemaphoreType.DMA,
        ],
    )(_emb_body)
    out = emb(xi, tpad, pos2)
    return out[:, :EMBED].reshape(B, Tcur, EMBED)
